# baseline (speedup 1.0000x reference)
import jax
import jax.numpy as jnp
from jax import lax
from jax.experimental import pallas as pl
from jax.experimental.pallas import tpu as pltpu

_B, _N, _INF, _OUTF = 4, 1024, 256, 256
_PB = 2


def _gat_body(inp_ref, adj_ref, w_ref, a_ref, out_ref):
    m = (adj_ref[...] > 0).astype(jnp.float32)
    a_top = a_ref[:_OUTF, :]
    a_bot = a_ref[_OUTF:, :]
    w = w_ref[...]
    row = lax.broadcasted_iota(jnp.int32, (_N, 1), 0)
    lane = lax.broadcasted_iota(jnp.int32, (1, _N), 1)
    for u in range(_PB):
        x = inp_ref[u]
        h = jnp.dot(x, w, preferred_element_type=jnp.float32)
        h = jnp.where(row == 0, 0.0, h)
        hT = lax.dot_general(w, x, (((0,), (1,)), ((), ())),
                             preferred_element_type=jnp.float32)
        hT = jnp.where(lane == 0, 0.0, hT)
        hp = pltpu.roll(h, 1, 0)
        hp = jnp.where(row == 0, 0.0, hp)
        gT = lax.dot_general(hp, m, (((0,), (0,)), ((), ())),
                             preferred_element_type=jnp.float32)
        vr = jnp.sum(hT * a_top + gT * a_bot, axis=0, keepdims=True)
        vr = jnp.where(lane == 0, 0.0, vr)
        vals = jnp.transpose(vr)
        ssum = jnp.sum(h, axis=0, keepdims=True)
        o = vals * ssum
        out_ref[u] = jnp.maximum(o, 0.2 * o)


def kernel(inp, adj, W, a):
    return pl.pallas_call(
        _gat_body,
        grid=(_B // _PB,),
        in_specs=[
            pl.BlockSpec((_PB, _N, _INF), lambda b: (b, 0, 0)),
            pl.BlockSpec((_N, _N), lambda b: (0, 0)),
            pl.BlockSpec((_INF, _OUTF), lambda b: (0, 0)),
            pl.BlockSpec((2 * _OUTF, _N), lambda b: (0, 0)),
        ],
        out_specs=pl.BlockSpec((_PB, _N, _OUTF), lambda b: (b, 0, 0)),
        out_shape=jax.ShapeDtypeStruct((_B, _N, _OUTF), jnp.float32),
        compiler_params=pltpu.CompilerParams(
            dimension_semantics=("arbitrary",),
        ),
    )(inp, adj, W, a)


# speedup vs baseline: 1.1017x; 1.1017x over previous
import jax
import jax.numpy as jnp
from jax import lax
from jax.experimental import pallas as pl
from jax.experimental.pallas import tpu as pltpu

_B, _N, _INF, _OUTF = 4, 1024, 256, 256
_PB = 2


def _gat_body(inp_ref, adj_ref, w_ref, a_ref, out_ref, at_s):
    s = pl.program_id(0)

    @pl.when(s == 0)
    def _():
        at_s[...] = jnp.transpose(a_ref[...])

    @pl.when(s > 0)
    def _steps():
        _compute(inp_ref, adj_ref, w_ref, out_ref, at_s)


def _compute(inp_ref, adj_ref, w_ref, out_ref, at_s):
    m = (adj_ref[...] > 0).astype(jnp.float32)
    at = at_s[...]
    row = lax.broadcasted_iota(jnp.int32, (_N, 1), 0)
    for u in range(_PB):
        x = inp_ref[u]
        h = jnp.dot(x, w_ref[...], preferred_element_type=jnp.float32)
        h = jnp.where(row == 0, 0.0, h)
        hp = pltpu.roll(h, 1, 0)
        hp = jnp.where(row == 0, 0.0, hp)
        g = lax.dot_general(m, hp, (((0,), (0,)), ((), ())),
                            preferred_element_type=jnp.float32)
        vals = (jnp.sum(h * at[:, :_OUTF], axis=1, keepdims=True)
                + jnp.sum(g * at[:, _OUTF:], axis=1, keepdims=True))
        vals = jnp.where(row == 0, 0.0, vals)
        ssum = jnp.sum(h, axis=0, keepdims=True)
        o = vals * ssum
        out_ref[u] = jnp.maximum(o, 0.2 * o)


def kernel(inp, adj, W, a):
    return pl.pallas_call(
        _gat_body,
        grid=(_B // _PB + 1,),
        in_specs=[
            pl.BlockSpec((_PB, _N, _INF), lambda b: (jnp.maximum(b - 1, 0), 0, 0)),
            pl.BlockSpec((_N, _N), lambda b: (0, 0)),
            pl.BlockSpec((_INF, _OUTF), lambda b: (0, 0)),
            pl.BlockSpec((2 * _OUTF, _N), lambda b: (0, 0)),
        ],
        out_specs=pl.BlockSpec((_PB, _N, _OUTF), lambda b: (jnp.maximum(b - 1, 0), 0, 0)),
        out_shape=jax.ShapeDtypeStruct((_B, _N, _OUTF), jnp.float32),
        scratch_shapes=[pltpu.VMEM((_N, 2 * _OUTF), jnp.float32)],
        compiler_params=pltpu.CompilerParams(
            dimension_semantics=("arbitrary",),
        ),
    )(inp, adj, W, a)
